# Initial kernel scaffold; baseline (speedup 1.0000x reference)
import functools

import jax
import jax.numpy as jnp
from jax import lax
from jax.experimental import pallas as pl
from jax.experimental.pallas import tpu as pltpu
from jax.experimental.pallas import tpu_sc as plsc

N = 10000
E = 320000
D = 128
NC = 2
NS = 16
NW = NC * NS
K = 128
CH = 80
EPW = CH * K
PAD = NW * EPW - E
NP = 10016
RPT = NP // NS

_mesh = plsc.VectorSubcoreMesh(core_axis_name="c", subcore_axis_name="s")


@functools.partial(
    pl.kernel,
    out_type=(
        jax.ShapeDtypeStruct((NC, NP, D), jnp.float32),
        jax.ShapeDtypeStruct((NC, NP, 16), jnp.float32),
    ),
    mesh=_mesh,
    scratch_types=[
        pltpu.VMEM((CH, K), jnp.int32),
        pltpu.VMEM((CH, K), jnp.int32),
        pltpu.VMEM((K, D), jnp.float32),
        pltpu.VMEM((K, 16), jnp.float32),
        pltpu.VMEM_SHARED((NP, D), jnp.float32),
        pltpu.VMEM_SHARED((NP, 16), jnp.float32),
        pltpu.SemaphoreType.DMA,
    ],
)
def _sc_gather_sum(x_hbm, src_hbm, dst_hbm, zn_hbm, zd_hbm, ones_hbm,
                   neigh_out, deg_out,
                   src_v, dst_v, rows_v, ones_v, sneigh, sdeg, sem):
    c = lax.axis_index("c")
    s = lax.axis_index("s")
    wid = c * NS + s

    r0 = s * RPT
    pltpu.sync_copy(zn_hbm.at[pl.ds(r0, RPT)], sneigh.at[pl.ds(r0, RPT)])
    pltpu.sync_copy(zd_hbm.at[pl.ds(r0, RPT)], sdeg.at[pl.ds(r0, RPT)])
    pltpu.sync_copy(ones_hbm, ones_v)
    pltpu.sync_copy(src_hbm.at[wid], src_v)
    pltpu.sync_copy(dst_hbm.at[wid], dst_v)
    plsc.subcore_barrier()

    def chunk(j, carry):
        pltpu.async_copy(x_hbm.at[src_v.at[j]], rows_v, sem).wait()
        pltpu.sync_copy(rows_v, sneigh.at[dst_v.at[j]], add=True)
        pltpu.sync_copy(ones_v, sdeg.at[dst_v.at[j]], add=True)
        return carry

    lax.fori_loop(0, CH, chunk, 0)
    plsc.subcore_barrier()

    pltpu.sync_copy(sneigh.at[pl.ds(r0, RPT)], neigh_out.at[c].at[pl.ds(r0, RPT)])
    pltpu.sync_copy(sdeg.at[pl.ds(r0, RPT)], deg_out.at[c].at[pl.ds(r0, RPT)])


_R = 2000


def _tc_body(x_ref, p0_ref, p1_ref, d0_ref, d1_ref, w1_ref, w2_ref, o_ref):
    deg = jnp.maximum(d0_ref[:, 0:1] + d1_ref[:, 0:1], 1.0)
    neigh = (p0_ref[...] + p1_ref[...]) / deg
    dn = (((1,), (1,)), ((), ()))
    o_ref[...] = (
        lax.dot_general(neigh, w1_ref[...], dn, preferred_element_type=jnp.float32)
        + lax.dot_general(x_ref[...], w2_ref[...], dn, preferred_element_type=jnp.float32)
    )


_tc_final = pl.pallas_call(
    _tc_body,
    grid=(N // _R,),
    in_specs=[
        pl.BlockSpec((_R, D), lambda i: (i, 0)),
        pl.BlockSpec((_R, D), lambda i: (i, 0)),
        pl.BlockSpec((_R, D), lambda i: (i, 0)),
        pl.BlockSpec((_R, 16), lambda i: (i, 0)),
        pl.BlockSpec((_R, 16), lambda i: (i, 0)),
        pl.BlockSpec((D, D), lambda i: (0, 0)),
        pl.BlockSpec((D, D), lambda i: (0, 0)),
    ],
    out_specs=pl.BlockSpec((_R, D), lambda i: (i, 0)),
    out_shape=jax.ShapeDtypeStruct((N, D), jnp.float32),
)


def kernel(x, edge_index, l, W1, W2):
    src = edge_index[0]
    dst = edge_index[1]
    src_p = jnp.concatenate([src, jnp.zeros((PAD,), jnp.int32)]).reshape(NW, CH, K)
    dst_p = jnp.concatenate([dst, jnp.full((PAD,), N, jnp.int32)]).reshape(NW, CH, K)
    zn = jnp.zeros((NP, D), jnp.float32)
    zd = jnp.zeros((NP, 16), jnp.float32)
    ones = jnp.ones((K, 16), jnp.float32)
    neigh2, deg2 = _sc_gather_sum(x, src_p, dst_p, zn, zd, ones)
    return _tc_final(x, neigh2[0, :N], neigh2[1, :N],
                     deg2[0, :N], deg2[1, :N], W1, W2)


# speedup vs baseline: 4.0724x; 4.0724x over previous
import functools

import jax
import jax.numpy as jnp
from jax import lax
from jax.experimental import pallas as pl
from jax.experimental.pallas import tpu as pltpu
from jax.experimental.pallas import tpu_sc as plsc

N = 10000
E = 320000
D = 128
DH = D // 2
NC = 2
NS = 16
K = 128
CH = 160
EPW = CH * K
PAD = NS * EPW - E
NP = 10112
RPT = NP // NS

_mesh = plsc.VectorSubcoreMesh(core_axis_name="c", subcore_axis_name="s")


@functools.partial(
    pl.kernel,
    out_type=(
        jax.ShapeDtypeStruct((NC, NP, DH), jnp.float32),
        jax.ShapeDtypeStruct((NC, NP, 16), jnp.float32),
    ),
    mesh=_mesh,
    compiler_params=pltpu.CompilerParams(use_tc_tiling_on_sc=False),
    scratch_types=[
        pltpu.VMEM((CH, K), jnp.int32),
        pltpu.VMEM((CH, K), jnp.int32),
        pltpu.VMEM((K, DH), jnp.float32),
        pltpu.VMEM((K, 16), jnp.float32),
        pltpu.VMEM_SHARED((NP, DH), jnp.float32),
        pltpu.VMEM_SHARED((NP, 16), jnp.float32),
        pltpu.SemaphoreType.DMA,
    ],
)
def _sc_gather_sum(xl_hbm, xr_hbm, src_hbm, dst_hbm, zn_hbm, zd_hbm, ones_hbm,
                   neigh_out, deg_out,
                   src_v, dst_v, rows_v, ones_v, sneigh, sdeg, sem):
    c = lax.axis_index("c")
    s = lax.axis_index("s")

    r0 = s * RPT
    pltpu.sync_copy(zn_hbm.at[pl.ds(r0, RPT)], sneigh.at[pl.ds(r0, RPT)])
    pltpu.sync_copy(zd_hbm.at[pl.ds(r0, RPT)], sdeg.at[pl.ds(r0, RPT)])
    pltpu.sync_copy(ones_hbm, ones_v)
    pltpu.sync_copy(src_hbm.at[s], src_v)
    pltpu.sync_copy(dst_hbm.at[s], dst_v)
    plsc.subcore_barrier()

    def do_chunks(x_hbm):
        def chunk(j, carry):
            pltpu.async_copy(x_hbm.at[src_v.at[j]], rows_v, sem).wait()
            pltpu.sync_copy(rows_v, sneigh.at[dst_v.at[j]], add=True)
            pltpu.sync_copy(ones_v, sdeg.at[dst_v.at[j]], add=True)
            return carry
        lax.fori_loop(0, CH, chunk, 0)

    @pl.when(c == 0)
    def _():
        do_chunks(xl_hbm)

    @pl.when(c == 1)
    def _():
        do_chunks(xr_hbm)

    plsc.subcore_barrier()

    pltpu.sync_copy(sneigh.at[pl.ds(r0, RPT)], neigh_out.at[c].at[pl.ds(r0, RPT)])
    pltpu.sync_copy(sdeg.at[pl.ds(r0, RPT)], deg_out.at[c].at[pl.ds(r0, RPT)])


_R = 2000


def _tc_body(x_ref, pl_ref, pr_ref, d0_ref, o_ref, w1_ref, w2_ref):
    deg = jnp.maximum(d0_ref[:, 0:1], 1.0)
    neigh = jnp.concatenate([pl_ref[...], pr_ref[...]], axis=1) / deg
    dn = (((1,), (1,)), ((), ()))
    o_ref[...] = (
        lax.dot_general(neigh, w1_ref[...], dn, preferred_element_type=jnp.float32)
        + lax.dot_general(x_ref[...], w2_ref[...], dn, preferred_element_type=jnp.float32)
    )


def _tc_final_call(x, pl_half, pr_half, d0, W1, W2):
    body = lambda x_ref, a, b, d, w1, w2, o: _tc_body(x_ref, a, b, d, o, w1, w2)
    return pl.pallas_call(
        body,
        grid=(N // _R,),
        in_specs=[
            pl.BlockSpec((_R, D), lambda i: (i, 0)),
            pl.BlockSpec((_R, DH), lambda i: (i, 0)),
            pl.BlockSpec((_R, DH), lambda i: (i, 0)),
            pl.BlockSpec((_R, 16), lambda i: (i, 0)),
            pl.BlockSpec((D, D), lambda i: (0, 0)),
            pl.BlockSpec((D, D), lambda i: (0, 0)),
        ],
        out_specs=pl.BlockSpec((_R, D), lambda i: (i, 0)),
        out_shape=jax.ShapeDtypeStruct((N, D), jnp.float32),
    )(x, pl_half, pr_half, d0, W1, W2)


def kernel(x, edge_index, l, W1, W2):
    src = edge_index[0]
    dst = edge_index[1]
    src_p = jnp.concatenate([src, jnp.zeros((PAD,), jnp.int32)]).reshape(NS, CH, K)
    dst_p = jnp.concatenate([dst, jnp.full((PAD,), N, jnp.int32)]).reshape(NS, CH, K)
    xl = x[:, :DH]
    xr = x[:, DH:]
    zn = jnp.zeros((NP, DH), jnp.float32)
    zd = jnp.zeros((NP, 16), jnp.float32)
    ones = jnp.ones((K, 16), jnp.float32)
    neigh2, deg2 = _sc_gather_sum(xl, xr, src_p, dst_p, zn, zd, ones)
    return _tc_final_call(x, neigh2[0, :N], neigh2[1, :N], deg2[0, :N], W1, W2)


# speedup vs baseline: 5.3305x; 1.3090x over previous
import functools

import jax
import jax.numpy as jnp
from jax import lax
from jax.experimental import pallas as pl
from jax.experimental.pallas import tpu as pltpu
from jax.experimental.pallas import tpu_sc as plsc

N = 10000
E = 320000
D = 128
DH = D // 2
NC = 2
NS = 16
K = 128
CH = 160
EPW = CH * K
PAD = NS * EPW - E
NP = 10112
RPT = NP // NS

_mesh = plsc.VectorSubcoreMesh(core_axis_name="c", subcore_axis_name="s")


@functools.partial(
    pl.kernel,
    out_type=(
        jax.ShapeDtypeStruct((NC, NP, DH), jnp.float32),
        jax.ShapeDtypeStruct((NC, NP, 16), jnp.float32),
    ),
    mesh=_mesh,
    compiler_params=pltpu.CompilerParams(use_tc_tiling_on_sc=False),
    scratch_types=[
        pltpu.VMEM((CH, K), jnp.int32),
        pltpu.VMEM((CH, K), jnp.int32),
        [pltpu.VMEM((K, DH), jnp.float32)] * 4,
        pltpu.VMEM((K, 16), jnp.float32),
        pltpu.VMEM_SHARED((NP, DH), jnp.float32),
        pltpu.VMEM_SHARED((NP, 16), jnp.float32),
        [pltpu.SemaphoreType.DMA] * 4,
    ],
)
def _sc_gather_sum(xl_hbm, xr_hbm, src_hbm, dst_hbm, zn_hbm, zd_hbm, ones_hbm,
                   neigh_out, deg_out,
                   src_v, dst_v, bufs, ones_v, sneigh, sdeg, sems):
    c = lax.axis_index("c")
    s = lax.axis_index("s")

    r0 = s * RPT
    pltpu.sync_copy(zn_hbm.at[pl.ds(r0, RPT)], sneigh.at[pl.ds(r0, RPT)])
    pltpu.sync_copy(zd_hbm.at[pl.ds(r0, RPT)], sdeg.at[pl.ds(r0, RPT)])
    pltpu.sync_copy(ones_hbm, ones_v)
    pltpu.sync_copy(src_hbm.at[s], src_v)
    pltpu.sync_copy(dst_hbm.at[s], dst_v)
    plsc.subcore_barrier()

    NB = 4
    GRP = CH // NB

    def do_chunks(x_hbm):
        for b in range(NB):
            pltpu.async_copy(x_hbm.at[src_v.at[b]], bufs[b], sems[b])

        def process(k, prefetch):
            for b in range(NB):
                j = k * NB + b
                pltpu.make_async_copy(x_hbm.at[src_v.at[j]], bufs[b], sems[b]).wait()
                pltpu.sync_copy(bufs[b], sneigh.at[dst_v.at[j]], add=True)
                pltpu.sync_copy(ones_v, sdeg.at[dst_v.at[j]], add=True)
                if prefetch:
                    pltpu.async_copy(x_hbm.at[src_v.at[j + NB]], bufs[b], sems[b])

        def group(k, carry):
            process(k, True)
            return carry

        lax.fori_loop(0, GRP - 1, group, 0)
        process(GRP - 1, False)

    @pl.when(c == 0)
    def _():
        do_chunks(xl_hbm)

    @pl.when(c == 1)
    def _():
        do_chunks(xr_hbm)

    plsc.subcore_barrier()

    pltpu.sync_copy(sneigh.at[pl.ds(r0, RPT)], neigh_out.at[c].at[pl.ds(r0, RPT)])
    pltpu.sync_copy(sdeg.at[pl.ds(r0, RPT)], deg_out.at[c].at[pl.ds(r0, RPT)])


_R = 2000


def _tc_body(x_ref, pl_ref, pr_ref, d0_ref, o_ref, w1_ref, w2_ref):
    deg = jnp.maximum(d0_ref[:, 0:1], 1.0)
    neigh = jnp.concatenate([pl_ref[...], pr_ref[...]], axis=1) / deg
    dn = (((1,), (1,)), ((), ()))
    o_ref[...] = (
        lax.dot_general(neigh, w1_ref[...], dn, preferred_element_type=jnp.float32)
        + lax.dot_general(x_ref[...], w2_ref[...], dn, preferred_element_type=jnp.float32)
    )


def _tc_final_call(x, pl_half, pr_half, d0, W1, W2):
    body = lambda x_ref, a, b, d, w1, w2, o: _tc_body(x_ref, a, b, d, o, w1, w2)
    return pl.pallas_call(
        body,
        grid=(N // _R,),
        in_specs=[
            pl.BlockSpec((_R, D), lambda i: (i, 0)),
            pl.BlockSpec((_R, DH), lambda i: (i, 0)),
            pl.BlockSpec((_R, DH), lambda i: (i, 0)),
            pl.BlockSpec((_R, 16), lambda i: (i, 0)),
            pl.BlockSpec((D, D), lambda i: (0, 0)),
            pl.BlockSpec((D, D), lambda i: (0, 0)),
        ],
        out_specs=pl.BlockSpec((_R, D), lambda i: (i, 0)),
        out_shape=jax.ShapeDtypeStruct((N, D), jnp.float32),
    )(x, pl_half, pr_half, d0, W1, W2)


def kernel(x, edge_index, l, W1, W2):
    src = edge_index[0]
    dst = edge_index[1]
    src_p = jnp.concatenate([src, jnp.zeros((PAD,), jnp.int32)]).reshape(NS, CH, K)
    dst_p = jnp.concatenate([dst, jnp.full((PAD,), N, jnp.int32)]).reshape(NS, CH, K)
    xl = x[:, :DH]
    xr = x[:, DH:]
    zn = jnp.zeros((NP, DH), jnp.float32)
    zd = jnp.zeros((NP, 16), jnp.float32)
    ones = jnp.ones((K, 16), jnp.float32)
    neigh2, deg2 = _sc_gather_sum(xl, xr, src_p, dst_p, zn, zd, ones)
    return _tc_final_call(x, neigh2[0, :N], neigh2[1, :N], deg2[0, :N], W1, W2)
